# baseline (speedup 1.0000x reference)
import functools

import jax
import jax.numpy as jnp
import numpy as np
from jax import lax
from jax.experimental import pallas as pl
from jax.experimental.pallas import tpu as pltpu
from jax.experimental.pallas import tpu_sc as plsc

N = 10000
NPAD = 10240
C = 256
NHEAD = 8
DIM = C // NHEAD
KNN_K = 16
EXTRA_K = 4
SCALE = DIM ** (-0.5)
ESLOT = 32
QB = 256
NBLK = NPAD // QB
QBK = 128
NEG = -1e30


def _proj_kernel(f_ref, w_ref, out_ref):
    out_ref[...] = jnp.dot(f_ref[...].astype(jnp.bfloat16),
                           w_ref[...].astype(jnp.bfloat16),
                           preferred_element_type=jnp.float32)


def _project(feat_pad, wqkv_t):
    return pl.pallas_call(
        _proj_kernel,
        grid=(NPAD // 512,),
        in_specs=[
            pl.BlockSpec((512, C), lambda i: (i, 0)),
            pl.BlockSpec((C, 3 * C), lambda i: (0, 0)),
        ],
        out_specs=pl.BlockSpec((512, 3 * C), lambda i: (i, 0)),
        out_shape=jax.ShapeDtypeStruct((NPAD, 3 * C), jnp.float32),
    )(feat_pad, wqkv_t)


def _knn_block_kernel(cq_ref, cT_ref, sqq_ref, sqc_ref, out_ref):
    n = cT_ref.shape[1]
    qc = jnp.dot(cq_ref[...].astype(jnp.bfloat16), cT_ref[...].astype(jnp.bfloat16),
                 preferred_element_type=jnp.float32)
    d = sqq_ref[...] - 2.0 * qc + sqc_ref[...]
    iota = jax.lax.broadcasted_iota(jnp.int32, (1, n), 1)
    cols = []
    for _ in range(KNN_K):
        m = jnp.min(d, axis=1, keepdims=True)
        idx = jnp.min(jnp.where(d == m, iota, n), axis=1, keepdims=True)
        cols.append(idx)
        d = jnp.where(iota == idx, jnp.inf, d)
    pad = jnp.zeros((cq_ref.shape[0], 128 - KNN_K), jnp.int32)
    out_ref[...] = jnp.concatenate(cols + [pad], axis=1)


def _knn_idx(coords_pad, coordsT, sq_col, sq_row):
    idx_pad = pl.pallas_call(
        _knn_block_kernel,
        grid=(NPAD // QBK,),
        in_specs=[
            pl.BlockSpec((QBK, 3), lambda i: (i, 0)),
            pl.BlockSpec((3, N), lambda i: (0, 0)),
            pl.BlockSpec((QBK, 1), lambda i: (i, 0)),
            pl.BlockSpec((1, N), lambda i: (0, 0)),
        ],
        out_specs=pl.BlockSpec((QBK, 128), lambda i: (i, 0)),
        out_shape=jax.ShapeDtypeStruct((NPAD, 128), jnp.int32),
    )(coords_pad[:, :3], coordsT, sq_col, sq_row)
    return idx_pad[:, :KNN_K]


NW = 32
EROWS = ESLOT * NPAD
PER_W = EROWS // NW
CH = 64


KXW = C + 128


def _sc_gather(kxmat, vmat, eidx):
    mesh = plsc.VectorSubcoreMesh(core_axis_name="c", subcore_axis_name="s")
    nch = PER_W // CH

    @functools.partial(
        pl.kernel, mesh=mesh,
        out_type=[
            jax.ShapeDtypeStruct((EROWS, KXW), jnp.float32),
            jax.ShapeDtypeStruct((EROWS, C), jnp.float32),
        ],
        scratch_types=[
            pltpu.VMEM((CH,), jnp.int32),
            pltpu.VMEM((CH,), jnp.int32),
            pltpu.VMEM((CH, KXW), jnp.float32),
            pltpu.VMEM((CH, KXW), jnp.float32),
            pltpu.VMEM((CH, C), jnp.float32),
            pltpu.VMEM((CH, C), jnp.float32),
            pltpu.SemaphoreType.DMA,
            pltpu.SemaphoreType.DMA,
            pltpu.SemaphoreType.DMA,
            pltpu.SemaphoreType.DMA,
        ],
    )
    def gather_k(kx_hbm, v_hbm, e_hbm, kg_hbm, vg_hbm,
                 idx0, idx1, kx0, kx1, v0, v1, g0, g1, s0, s1):
        wid = lax.axis_index("s") * 2 + lax.axis_index("c")
        base = wid * PER_W
        sets = ((idx0, kx0, v0, g0, s0), (idx1, kx1, v1, g1, s1))

        def step(j, mine, other):
            idxa, kxa, va, ga, sa = mine
            idxb, kxb, vb, gb, sb = other
            off = base + j * CH

            @pl.when(j >= 2)
            def _():
                pltpu.make_async_copy(kxa, kg_hbm.at[pl.ds(off, CH)], sa).wait()
                pltpu.make_async_copy(va, vg_hbm.at[pl.ds(off, CH)], sa).wait()

            pltpu.sync_copy(e_hbm.at[pl.ds(off, CH)], idxa)
            pltpu.async_copy(kx_hbm.at[idxa], kxa, ga)
            pltpu.async_copy(v_hbm.at[idxa], va, ga)

            @pl.when(j >= 1)
            def _():
                offp = off - CH
                pltpu.make_async_copy(kx_hbm.at[idxb], kxb, gb).wait()
                pltpu.make_async_copy(v_hbm.at[idxb], vb, gb).wait()
                pltpu.async_copy(kxb, kg_hbm.at[pl.ds(offp, CH)], sb)
                pltpu.async_copy(vb, vg_hbm.at[pl.ds(offp, CH)], sb)

        def body(i, carry):
            step(2 * i, sets[0], sets[1])
            step(2 * i + 1, sets[1], sets[0])
            return carry

        lax.fori_loop(0, nch // 2, body, 0)
        offl = base + (nch - 1) * CH
        pltpu.make_async_copy(kx_hbm.at[idx1], kx1, g1).wait()
        pltpu.make_async_copy(v_hbm.at[idx1], v1, g1).wait()
        pltpu.sync_copy(kx1, kg_hbm.at[pl.ds(offl, CH)])
        pltpu.sync_copy(v1, vg_hbm.at[pl.ds(offl, CH)])
        pltpu.make_async_copy(kx0, kg_hbm.at[pl.ds(base, CH)], s0).wait()
        pltpu.make_async_copy(v0, vg_hbm.at[pl.ds(base, CH)], s0).wait()

    return gather_k(kxmat, vmat, eidx)


def _attn_kernel(q_ref, f_ref, cq_ref, kg_ref, vg_ref, mask_ref,
                 wpos_ref, bpos_ref, bd_ref, rept_ref, wot_ref, g_ref, b_ref,
                 out_ref):
    q = q_ref[...]
    cq = cq_ref[...]
    wpos_b = wpos_ref[...].astype(jnp.bfloat16)
    bd = bd_ref[...]
    hi = jax.lax.Precision.HIGHEST
    ss = []
    m = jnp.full((QB, NHEAD), -jnp.inf, jnp.float32)
    for t in range(ESLOT):
        prod = q * kg_ref[t][:, :C]
        s = jnp.dot(prod, bd, precision=hi) * SCALE
        rel = cq - kg_ref[t][:, C:C + 16]
        pe = jnp.dot(rel.astype(jnp.bfloat16), wpos_b,
                     preferred_element_type=jnp.float32)
        pe = pe + bpos_ref[...]
        s = s + pe + mask_ref[t]
        ss.append(s)
        m = jnp.maximum(m, s)
    acc = jnp.zeros((QB, C), jnp.float32)
    den = jnp.zeros((QB, NHEAD), jnp.float32)
    rept = rept_ref[...]
    for t in range(ESLOT):
        ex = jnp.exp(ss[t] - m)
        den = den + ex
        wex = jnp.dot(ex, rept, precision=hi)
        acc = acc + vg_ref[t] * wex
    deninv = jnp.dot(den, rept, precision=hi)
    out = acc / deninv
    of = jnp.dot(out.astype(jnp.bfloat16), wot_ref[...].astype(jnp.bfloat16),
                 preferred_element_type=jnp.float32)
    resid = of + f_ref[...]
    mu = jnp.mean(resid, axis=-1, keepdims=True)
    var = jnp.mean((resid - mu) ** 2, axis=-1, keepdims=True)
    normed = (resid - mu) / jnp.sqrt(var + 1e-5)
    out_ref[...] = normed * g_ref[...] + b_ref[...]


def _attention(q_pad, feat_pad, coords_pad, kg3, vg3, mask3,
               wpos_pad, bpos2, bd, rept, wot, gamma2, beta2):
    return pl.pallas_call(
        _attn_kernel,
        grid=(NBLK,),
        in_specs=[
            pl.BlockSpec((QB, C), lambda i: (i, 0)),
            pl.BlockSpec((QB, C), lambda i: (i, 0)),
            pl.BlockSpec((QB, 16), lambda i: (i, 0)),
            pl.BlockSpec((ESLOT, QB, KXW), lambda i: (0, i, 0)),
            pl.BlockSpec((ESLOT, QB, C), lambda i: (0, i, 0)),
            pl.BlockSpec((ESLOT, QB, NHEAD), lambda i: (0, i, 0)),
            pl.BlockSpec((16, NHEAD), lambda i: (0, 0)),
            pl.BlockSpec((1, NHEAD), lambda i: (0, 0)),
            pl.BlockSpec((C, NHEAD), lambda i: (0, 0)),
            pl.BlockSpec((NHEAD, C), lambda i: (0, 0)),
            pl.BlockSpec((C, C), lambda i: (0, 0)),
            pl.BlockSpec((1, C), lambda i: (0, 0)),
            pl.BlockSpec((1, C), lambda i: (0, 0)),
        ],
        out_specs=pl.BlockSpec((QB, C), lambda i: (i, 0)),
        out_shape=jax.ShapeDtypeStruct((NPAD, C), jnp.float32),
    )(q_pad, feat_pad, coords_pad, kg3, vg3, mask3,
      wpos_pad, bpos2, bd, rept, wot, gamma2, beta2)


def _edge_tables():
    ek = jax.random.key(12345)
    extra_q = jax.random.randint(jax.random.fold_in(ek, 0), (N * EXTRA_K,), 0, N,
                                 dtype=jnp.int32)
    extra_n = jax.random.randint(jax.random.fold_in(ek, 1), (N * EXTRA_K,), 0, N,
                                 dtype=jnp.int32)
    order = jnp.argsort(extra_q, stable=True)
    sq_ = extra_q[order]
    sn_ = extra_n[order]
    start = jnp.searchsorted(sq_, jnp.arange(N, dtype=jnp.int32), side="left")
    pos = jnp.arange(N * EXTRA_K, dtype=jnp.int32) - start[sq_]
    ext_nbr = jnp.zeros((NPAD, KNN_K), jnp.int32).at[sq_, pos].set(sn_)
    cnt = jnp.zeros((NPAD,), jnp.int32).at[sq_].add(1)
    tidx = jnp.arange(ESLOT, dtype=jnp.int32)[:, None]
    valid = (tidx < KNN_K + cnt[None, :]) & (jnp.arange(NPAD)[None, :] < N)
    mask3 = jnp.where(valid, 0.0, NEG).astype(jnp.float32)
    mask3 = jnp.broadcast_to(mask3[:, :, None], (ESLOT, NPAD, NHEAD))
    return ext_nbr, mask3


def kernel(features, coords, Wq, Wk, Wv, Wo, Wpos, bpos, gamma, beta):
    n = features.shape[0]
    pad = NPAD - n
    feat_pad = jnp.pad(features, ((0, pad), (0, 0)))
    coords_pad = jnp.pad(coords, ((0, pad), (0, 13)))
    sq = jnp.sum(coords * coords, axis=1)
    sq_col = jnp.pad(sq, (0, pad)).reshape(NPAD, 1)
    sq_row = sq.reshape(1, N)
    coordsT = coords.T

    wqkv_t = jnp.concatenate([Wq.T, Wk.T, Wv.T], axis=1)
    qkv = _project(feat_pad, wqkv_t)
    q_pad = qkv[:, :C]
    k_pad = qkv[:, C:2 * C]
    v_pad = qkv[:, 2 * C:]

    knn = _knn_idx(coords_pad, coordsT, sq_col, sq_row)

    ext_nbr, mask3 = _edge_tables()
    etab = jnp.concatenate([knn.T, ext_nbr.T], axis=0)
    eidx = etab.reshape(EROWS)
    cpad128 = jnp.pad(coords_pad, ((0, 0), (0, 112)))
    kx_pad = jnp.concatenate([k_pad, cpad128], axis=1)
    kg, vg = _sc_gather(kx_pad, v_pad, eidx)
    kg3 = kg.reshape(ESLOT, NPAD, KXW)
    vg3 = vg.reshape(ESLOT, NPAD, C)

    wpos_pad = jnp.pad(Wpos.T, ((0, 13), (0, 0)))
    bd = (jnp.arange(C)[:, None] // DIM ==
          jnp.arange(NHEAD)[None, :]).astype(jnp.float32)
    rept = bd.T
    normed = _attention(q_pad, feat_pad, coords_pad, kg3, vg3, mask3,
                        wpos_pad, bpos.reshape(1, NHEAD), bd, rept, Wo.T,
                        gamma.reshape(1, C), beta.reshape(1, C))
    return normed[:n]


# speedup vs baseline: 1.0389x; 1.0389x over previous
import functools

import jax
import jax.numpy as jnp
import numpy as np
from jax import lax
from jax.experimental import pallas as pl
from jax.experimental.pallas import tpu as pltpu
from jax.experimental.pallas import tpu_sc as plsc

N = 10000
NPAD = 10240
C = 256
NHEAD = 8
DIM = C // NHEAD
KNN_K = 16
EXTRA_K = 4
SCALE = DIM ** (-0.5)
ESLOT = 32
QB = 256
NBLK = NPAD // QB
QBK = 128
NEG = -1e30


def _proj_kernel(f_ref, w_ref, out_ref):
    out_ref[...] = jnp.dot(f_ref[...].astype(jnp.bfloat16),
                           w_ref[...].astype(jnp.bfloat16),
                           preferred_element_type=jnp.float32)


def _project(feat_pad, wqkv_t):
    return pl.pallas_call(
        _proj_kernel,
        grid=(NPAD // 512,),
        in_specs=[
            pl.BlockSpec((512, C), lambda i: (i, 0)),
            pl.BlockSpec((C, 3 * C), lambda i: (0, 0)),
        ],
        out_specs=pl.BlockSpec((512, 3 * C), lambda i: (i, 0)),
        out_shape=jax.ShapeDtypeStruct((NPAD, 3 * C), jnp.float32),
    )(feat_pad, wqkv_t)


def _knn_block_kernel(cq_ref, cT_ref, sqq_ref, sqc_ref, out_ref):
    n = cT_ref.shape[1]
    qc = jnp.dot(cq_ref[...].astype(jnp.bfloat16), cT_ref[...].astype(jnp.bfloat16),
                 preferred_element_type=jnp.float32)
    d = sqq_ref[...] - 2.0 * qc + sqc_ref[...]
    iota = jax.lax.broadcasted_iota(jnp.int32, (1, n), 1)
    cols = []
    for _ in range(KNN_K):
        m = jnp.min(d, axis=1, keepdims=True)
        idx = jnp.min(jnp.where(d == m, iota, n), axis=1, keepdims=True)
        cols.append(idx)
        d = jnp.where(iota == idx, jnp.inf, d)
    pad = jnp.zeros((cq_ref.shape[0], 128 - KNN_K), jnp.int32)
    out_ref[...] = jnp.concatenate(cols + [pad], axis=1)


def _knn_idx(coords_pad, coordsT, sq_col, sq_row):
    idx_pad = pl.pallas_call(
        _knn_block_kernel,
        grid=(NPAD // QBK,),
        in_specs=[
            pl.BlockSpec((QBK, 3), lambda i: (i, 0)),
            pl.BlockSpec((3, N), lambda i: (0, 0)),
            pl.BlockSpec((QBK, 1), lambda i: (i, 0)),
            pl.BlockSpec((1, N), lambda i: (0, 0)),
        ],
        out_specs=pl.BlockSpec((QBK, 128), lambda i: (i, 0)),
        out_shape=jax.ShapeDtypeStruct((NPAD, 128), jnp.int32),
    )(coords_pad[:, :3], coordsT, sq_col, sq_row)
    return idx_pad[:, :KNN_K]


NW = 32
EROWS = ESLOT * NPAD
PER_W = EROWS // NW
CH = 64


FW = 2 * C + 128
HROWS = KNN_K * NPAD
PER_W2 = HROWS // NW


def _sc_gather_half(fused, eidx):
    mesh = plsc.VectorSubcoreMesh(core_axis_name="c", subcore_axis_name="s")
    nch = PER_W2 // CH

    @functools.partial(
        pl.kernel, mesh=mesh,
        out_type=jax.ShapeDtypeStruct((HROWS, FW), jnp.float32),
        scratch_types=[
            pltpu.VMEM((CH,), jnp.int32),
            pltpu.VMEM((CH,), jnp.int32),
            pltpu.VMEM((CH, FW), jnp.float32),
            pltpu.VMEM((CH, FW), jnp.float32),
            pltpu.SemaphoreType.DMA,
            pltpu.SemaphoreType.DMA,
            pltpu.SemaphoreType.DMA,
            pltpu.SemaphoreType.DMA,
        ],
    )
    def gather_k(t_hbm, e_hbm, out_hbm, idx0, idx1, b0, b1, g0, g1, s0, s1):
        wid = lax.axis_index("s") * 2 + lax.axis_index("c")
        base = wid * PER_W2
        sets = ((idx0, b0, g0, s0), (idx1, b1, g1, s1))

        def step(j, mine, other):
            idxa, ba, ga, sa = mine
            idxb, bb, gb, sb = other
            off = base + j * CH

            @pl.when(j >= 2)
            def _():
                pltpu.make_async_copy(ba, out_hbm.at[pl.ds(off, CH)], sa).wait()

            pltpu.sync_copy(e_hbm.at[pl.ds(off, CH)], idxa)
            pltpu.async_copy(t_hbm.at[idxa], ba, ga)

            @pl.when(j >= 1)
            def _():
                offp = off - CH
                pltpu.make_async_copy(t_hbm.at[idxb], bb, gb).wait()
                pltpu.async_copy(bb, out_hbm.at[pl.ds(offp, CH)], sb)

        def body(i, carry):
            step(2 * i, sets[0], sets[1])
            step(2 * i + 1, sets[1], sets[0])
            return carry

        lax.fori_loop(0, nch // 2, body, 0)
        offl = base + (nch - 1) * CH
        pltpu.make_async_copy(t_hbm.at[idx1], b1, g1).wait()
        pltpu.sync_copy(b1, out_hbm.at[pl.ds(offl, CH)])
        pltpu.make_async_copy(b0, out_hbm.at[pl.ds(base, CH)], s0).wait()

    return gather_k(fused, eidx)


def _attn_kernel(q_ref, f_ref, cq_ref, ka_ref, kb_ref, mask_ref,
                 wpos_ref, bpos_ref, bd_ref, rept_ref, wot_ref, g_ref, b_ref,
                 out_ref):
    q = q_ref[...]
    cq = cq_ref[...]
    wpos_b = wpos_ref[...].astype(jnp.bfloat16)
    bd = bd_ref[...]
    hi = jax.lax.Precision.HIGHEST

    def slab(t):
        r = ka_ref if t < KNN_K else kb_ref
        return r[t % KNN_K]

    ss = []
    m = jnp.full((QB, NHEAD), -jnp.inf, jnp.float32)
    for t in range(ESLOT):
        row = slab(t)
        prod = q * row[:, :C]
        s = jnp.dot(prod, bd, precision=hi) * SCALE
        rel = cq - row[:, 2 * C:2 * C + 16]
        pe = jnp.dot(rel.astype(jnp.bfloat16), wpos_b,
                     preferred_element_type=jnp.float32)
        pe = pe + bpos_ref[...]
        s = s + pe + mask_ref[t]
        ss.append(s)
        m = jnp.maximum(m, s)
    acc = jnp.zeros((QB, C), jnp.float32)
    den = jnp.zeros((QB, NHEAD), jnp.float32)
    rept = rept_ref[...]
    for t in range(ESLOT):
        ex = jnp.exp(ss[t] - m)
        den = den + ex
        wex = jnp.dot(ex, rept, precision=hi)
        acc = acc + slab(t)[:, C:2 * C] * wex
    deninv = jnp.dot(den, rept, precision=hi)
    out = acc / deninv
    of = jnp.dot(out.astype(jnp.bfloat16), wot_ref[...].astype(jnp.bfloat16),
                 preferred_element_type=jnp.float32)
    resid = of + f_ref[...]
    mu = jnp.mean(resid, axis=-1, keepdims=True)
    var = jnp.mean((resid - mu) ** 2, axis=-1, keepdims=True)
    normed = (resid - mu) / jnp.sqrt(var + 1e-5)
    out_ref[...] = normed * g_ref[...] + b_ref[...]


def _attention(q_pad, feat_pad, coords_pad, ka3, kb3, mask3,
               wpos_pad, bpos2, bd, rept, wot, gamma2, beta2):
    return pl.pallas_call(
        _attn_kernel,
        grid=(NBLK,),
        in_specs=[
            pl.BlockSpec((QB, C), lambda i: (i, 0)),
            pl.BlockSpec((QB, C), lambda i: (i, 0)),
            pl.BlockSpec((QB, 16), lambda i: (i, 0)),
            pl.BlockSpec((KNN_K, QB, FW), lambda i: (0, i, 0)),
            pl.BlockSpec((KNN_K, QB, FW), lambda i: (0, i, 0)),
            pl.BlockSpec((ESLOT, QB, NHEAD), lambda i: (0, i, 0)),
            pl.BlockSpec((16, NHEAD), lambda i: (0, 0)),
            pl.BlockSpec((1, NHEAD), lambda i: (0, 0)),
            pl.BlockSpec((C, NHEAD), lambda i: (0, 0)),
            pl.BlockSpec((NHEAD, C), lambda i: (0, 0)),
            pl.BlockSpec((C, C), lambda i: (0, 0)),
            pl.BlockSpec((1, C), lambda i: (0, 0)),
            pl.BlockSpec((1, C), lambda i: (0, 0)),
        ],
        out_specs=pl.BlockSpec((QB, C), lambda i: (i, 0)),
        out_shape=jax.ShapeDtypeStruct((NPAD, C), jnp.float32),
    )(q_pad, feat_pad, coords_pad, ka3, kb3, mask3,
      wpos_pad, bpos2, bd, rept, wot, gamma2, beta2)


def _edge_tables():
    ek = jax.random.key(12345)
    extra_q = jax.random.randint(jax.random.fold_in(ek, 0), (N * EXTRA_K,), 0, N,
                                 dtype=jnp.int32)
    extra_n = jax.random.randint(jax.random.fold_in(ek, 1), (N * EXTRA_K,), 0, N,
                                 dtype=jnp.int32)
    order = jnp.argsort(extra_q, stable=True)
    sq_ = extra_q[order]
    sn_ = extra_n[order]
    start = jnp.searchsorted(sq_, jnp.arange(N, dtype=jnp.int32), side="left")
    pos = jnp.arange(N * EXTRA_K, dtype=jnp.int32) - start[sq_]
    ext_nbr = jnp.zeros((NPAD, KNN_K), jnp.int32).at[sq_, pos].set(sn_)
    cnt = jnp.zeros((NPAD,), jnp.int32).at[sq_].add(1)
    tidx = jnp.arange(ESLOT, dtype=jnp.int32)[:, None]
    valid = (tidx < KNN_K + cnt[None, :]) & (jnp.arange(NPAD)[None, :] < N)
    mask3 = jnp.where(valid, 0.0, NEG).astype(jnp.float32)
    mask3 = jnp.broadcast_to(mask3[:, :, None], (ESLOT, NPAD, NHEAD))
    return ext_nbr, mask3


def kernel(features, coords, Wq, Wk, Wv, Wo, Wpos, bpos, gamma, beta):
    n = features.shape[0]
    pad = NPAD - n
    feat_pad = jnp.pad(features, ((0, pad), (0, 0)))
    coords_pad = jnp.pad(coords, ((0, pad), (0, 13)))
    sq = jnp.sum(coords * coords, axis=1)
    sq_col = jnp.pad(sq, (0, pad)).reshape(NPAD, 1)
    sq_row = sq.reshape(1, N)
    coordsT = coords.T

    wqkv_t = jnp.concatenate([Wq.T, Wk.T, Wv.T], axis=1)
    qkv = _project(feat_pad, wqkv_t)
    q_pad = qkv[:, :C]
    k_pad = qkv[:, C:2 * C]
    v_pad = qkv[:, 2 * C:]

    knn = _knn_idx(coords_pad, coordsT, sq_col, sq_row)

    ext_nbr, mask3 = _edge_tables()
    cpad128 = jnp.pad(coords_pad, ((0, 0), (0, 112)))
    fused = jnp.concatenate([k_pad, v_pad, cpad128], axis=1)
    eidx_ext = ext_nbr.T.reshape(HROWS)
    kb = _sc_gather_half(fused, eidx_ext)
    eidx_knn = knn.T.reshape(HROWS)
    ka = _sc_gather_half(fused, eidx_knn)
    ka3 = ka.reshape(KNN_K, NPAD, FW)
    kb3 = kb.reshape(KNN_K, NPAD, FW)

    wpos_pad = jnp.pad(Wpos.T, ((0, 13), (0, 0)))
    bd = (jnp.arange(C)[:, None] // DIM ==
          jnp.arange(NHEAD)[None, :]).astype(jnp.float32)
    rept = bd.T
    normed = _attention(q_pad, feat_pad, coords_pad, ka3, kb3, mask3,
                        wpos_pad, bpos.reshape(1, NHEAD), bd, rept, Wo.T,
                        gamma.reshape(1, C), beta.reshape(1, C))
    return normed[:n]


# speedup vs baseline: 1.0400x; 1.0011x over previous
import functools

import jax
import jax.numpy as jnp
import numpy as np
from jax import lax
from jax.experimental import pallas as pl
from jax.experimental.pallas import tpu as pltpu
from jax.experimental.pallas import tpu_sc as plsc

N = 10000
NPAD = 10240
C = 256
NHEAD = 8
DIM = C // NHEAD
KNN_K = 16
EXTRA_K = 4
SCALE = DIM ** (-0.5)
ESLOT = 32
QB = 256
NBLK = NPAD // QB
QBK = 128
NEG = -1e30


def _proj_kernel(f_ref, w_ref, out_ref):
    out_ref[...] = jnp.dot(f_ref[...].astype(jnp.bfloat16),
                           w_ref[...].astype(jnp.bfloat16),
                           preferred_element_type=jnp.float32)


def _project(feat_pad, wqkv_t):
    return pl.pallas_call(
        _proj_kernel,
        grid=(NPAD // 512,),
        in_specs=[
            pl.BlockSpec((512, C), lambda i: (i, 0)),
            pl.BlockSpec((C, 3 * C), lambda i: (0, 0)),
        ],
        out_specs=pl.BlockSpec((512, 3 * C), lambda i: (i, 0)),
        out_shape=jax.ShapeDtypeStruct((NPAD, 3 * C), jnp.float32),
    )(feat_pad, wqkv_t)


def _knn_block_kernel(cq_ref, cT_ref, sqq_ref, sqc_ref, out_ref):
    n = cT_ref.shape[1]
    qc = jnp.dot(cq_ref[...].astype(jnp.bfloat16), cT_ref[...].astype(jnp.bfloat16),
                 preferred_element_type=jnp.float32)
    d = sqq_ref[...] - 2.0 * qc + sqc_ref[...]
    iota = jax.lax.broadcasted_iota(jnp.int32, (1, n), 1)
    cols = []
    for _ in range(KNN_K):
        m = jnp.min(d, axis=1, keepdims=True)
        idx = jnp.min(jnp.where(d == m, iota, n), axis=1, keepdims=True)
        cols.append(idx)
        d = jnp.where(iota == idx, jnp.inf, d)
    pad = jnp.zeros((cq_ref.shape[0], 128 - KNN_K), jnp.int32)
    out_ref[...] = jnp.concatenate(cols + [pad], axis=1)


def _knn_idx(coords_pad, coordsT, sq_col, sq_row):
    idx_pad = pl.pallas_call(
        _knn_block_kernel,
        grid=(NPAD // QBK,),
        in_specs=[
            pl.BlockSpec((QBK, 3), lambda i: (i, 0)),
            pl.BlockSpec((3, N), lambda i: (0, 0)),
            pl.BlockSpec((QBK, 1), lambda i: (i, 0)),
            pl.BlockSpec((1, N), lambda i: (0, 0)),
        ],
        out_specs=pl.BlockSpec((QBK, 128), lambda i: (i, 0)),
        out_shape=jax.ShapeDtypeStruct((NPAD, 128), jnp.int32),
    )(coords_pad[:, :3], coordsT, sq_col, sq_row)
    return idx_pad[:, :KNN_K]


NW = 32
EROWS = ESLOT * NPAD
PER_W = EROWS // NW
CH = 64


FW = 2 * C + 128
HROWS = KNN_K * NPAD
PER_W2 = HROWS // NW


def _sc_gather_half(fused, eidx):
    mesh = plsc.VectorSubcoreMesh(core_axis_name="c", subcore_axis_name="s")
    nch = PER_W2 // CH

    @functools.partial(
        pl.kernel, mesh=mesh,
        out_type=jax.ShapeDtypeStruct((HROWS, FW), jnp.float32),
        scratch_types=[
            pltpu.VMEM((CH,), jnp.int32),
            pltpu.VMEM((CH,), jnp.int32),
            pltpu.VMEM((CH, FW), jnp.float32),
            pltpu.VMEM((CH, FW), jnp.float32),
            pltpu.SemaphoreType.DMA,
            pltpu.SemaphoreType.DMA,
            pltpu.SemaphoreType.DMA,
            pltpu.SemaphoreType.DMA,
        ],
    )
    def gather_k(t_hbm, e_hbm, out_hbm, idx0, idx1, b0, b1, g0, g1, s0, s1):
        wid = lax.axis_index("s") * 2 + lax.axis_index("c")
        base = wid * PER_W2
        sets = ((idx0, b0, g0, s0), (idx1, b1, g1, s1))

        def step(j, mine, other):
            idxa, ba, ga, sa = mine
            idxb, bb, gb, sb = other
            off = base + j * CH

            @pl.when(j >= 2)
            def _():
                pltpu.make_async_copy(ba, out_hbm.at[pl.ds(off, CH)], sa).wait()

            pltpu.sync_copy(e_hbm.at[pl.ds(off, CH)], idxa)
            pltpu.async_copy(t_hbm.at[idxa], ba, ga)

            @pl.when(j >= 1)
            def _():
                offp = off - CH
                pltpu.make_async_copy(t_hbm.at[idxb], bb, gb).wait()
                pltpu.async_copy(bb, out_hbm.at[pl.ds(offp, CH)], sb)

        def body(i, carry):
            step(2 * i, sets[0], sets[1])
            step(2 * i + 1, sets[1], sets[0])
            return carry

        lax.fori_loop(0, nch // 2, body, 0)
        offl = base + (nch - 1) * CH
        pltpu.make_async_copy(t_hbm.at[idx1], b1, g1).wait()
        pltpu.sync_copy(b1, out_hbm.at[pl.ds(offl, CH)])
        pltpu.make_async_copy(b0, out_hbm.at[pl.ds(base, CH)], s0).wait()

    return gather_k(fused, eidx)


def _attn_kernel(q_ref, f_ref, cq_ref, ka_ref, kb_ref, mask_ref,
                 wpos_ref, bpos_ref, bd_ref, rept_ref, wot_ref, g_ref, b_ref,
                 out_ref):
    q = q_ref[...]
    cq = cq_ref[...]
    wpos_b = wpos_ref[...].astype(jnp.bfloat16)
    bd = bd_ref[...]
    hi = jax.lax.Precision.HIGHEST

    def slab(t):
        r = ka_ref if t < KNN_K else kb_ref
        return r[t % KNN_K]

    ss = []
    m = jnp.full((QB, NHEAD), -jnp.inf, jnp.float32)
    for t in range(ESLOT):
        row = slab(t)
        prod = q * row[:, :C]
        s = jnp.dot(prod, bd, precision=hi) * SCALE
        rel = cq - row[:, 2 * C:2 * C + 16]
        pe = jnp.dot(rel.astype(jnp.bfloat16), wpos_b,
                     preferred_element_type=jnp.float32)
        pe = pe + bpos_ref[...]
        s = s + pe + mask_ref[t]
        ss.append(s)
        m = jnp.maximum(m, s)
    acc = jnp.zeros((QB, C), jnp.float32)
    den = jnp.zeros((QB, NHEAD), jnp.float32)
    rept = rept_ref[...]
    for t in range(ESLOT):
        ex = jnp.exp(ss[t] - m)
        den = den + ex
        wex = jnp.dot(ex, rept, precision=hi)
        acc = acc + slab(t)[:, C:2 * C] * wex
    deninv = jnp.dot(den, rept, precision=hi)
    out = acc / deninv
    of = jnp.dot(out.astype(jnp.bfloat16), wot_ref[...].astype(jnp.bfloat16),
                 preferred_element_type=jnp.float32)
    resid = of + f_ref[...]
    mu = jnp.mean(resid, axis=-1, keepdims=True)
    var = jnp.mean((resid - mu) ** 2, axis=-1, keepdims=True)
    normed = (resid - mu) / jnp.sqrt(var + 1e-5)
    out_ref[...] = normed * g_ref[...] + b_ref[...]


def _attention(q_pad, feat_pad, coords_pad, ka3, kb3, mask3,
               wpos_pad, bpos2, bd, rept, wot, gamma2, beta2):
    return pl.pallas_call(
        _attn_kernel,
        grid=(NBLK,),
        in_specs=[
            pl.BlockSpec((QB, C), lambda i: (i, 0)),
            pl.BlockSpec((QB, C), lambda i: (i, 0)),
            pl.BlockSpec((QB, 16), lambda i: (i, 0)),
            pl.BlockSpec((KNN_K, QB, FW), lambda i: (0, i, 0)),
            pl.BlockSpec((KNN_K, QB, FW), lambda i: (0, i, 0)),
            pl.BlockSpec((ESLOT, QB, NHEAD), lambda i: (0, i, 0)),
            pl.BlockSpec((16, NHEAD), lambda i: (0, 0)),
            pl.BlockSpec((1, NHEAD), lambda i: (0, 0)),
            pl.BlockSpec((C, NHEAD), lambda i: (0, 0)),
            pl.BlockSpec((NHEAD, C), lambda i: (0, 0)),
            pl.BlockSpec((C, C), lambda i: (0, 0)),
            pl.BlockSpec((1, C), lambda i: (0, 0)),
            pl.BlockSpec((1, C), lambda i: (0, 0)),
        ],
        out_specs=pl.BlockSpec((QB, C), lambda i: (i, 0)),
        out_shape=jax.ShapeDtypeStruct((NPAD, C), jnp.float32),
    )(q_pad, feat_pad, coords_pad, ka3, kb3, mask3,
      wpos_pad, bpos2, bd, rept, wot, gamma2, beta2)


def _edge_tables():
    ek = jax.random.key(12345)
    extra_q = jax.random.randint(jax.random.fold_in(ek, 0), (N * EXTRA_K,), 0, N,
                                 dtype=jnp.int32)
    extra_n = jax.random.randint(jax.random.fold_in(ek, 1), (N * EXTRA_K,), 0, N,
                                 dtype=jnp.int32)
    order = jnp.argsort(extra_q, stable=True)
    sq_ = extra_q[order]
    sn_ = extra_n[order]
    start = jnp.searchsorted(sq_, jnp.arange(N, dtype=jnp.int32), side="left")
    pos = jnp.arange(N * EXTRA_K, dtype=jnp.int32) - start[sq_]
    ext_nbr = jnp.zeros((NPAD, KNN_K), jnp.int32).at[sq_, pos].set(sn_)
    cnt = jnp.zeros((NPAD,), jnp.int32).at[sq_].add(1)
    tidx = jnp.arange(ESLOT, dtype=jnp.int32)[:, None]
    valid = (tidx < KNN_K + cnt[None, :]) & (jnp.arange(NPAD)[None, :] < N)
    mask3 = jnp.where(valid, 0.0, NEG).astype(jnp.float32)
    mask3 = jnp.broadcast_to(mask3[:, :, None], (ESLOT, NPAD, NHEAD))
    return ext_nbr, mask3


def kernel(features, coords, Wq, Wk, Wv, Wo, Wpos, bpos, gamma, beta):
    n = features.shape[0]
    pad = NPAD - n
    feat_pad = jnp.pad(features, ((0, pad), (0, 0)))
    coords_pad = jnp.pad(coords, ((0, pad), (0, 13)))
    sq = jnp.sum(coords * coords, axis=1)
    sq_col = jnp.pad(sq, (0, pad)).reshape(NPAD, 1)
    sq_row = sq.reshape(1, N)
    coordsT = coords.T

    wqkv_t = jnp.concatenate([Wq.T, Wk.T, Wv.T], axis=1)
    qkv = _project(feat_pad, wqkv_t)
    q_pad = qkv[:, :C]
    k_pad = qkv[:, C:2 * C]
    v_pad = qkv[:, 2 * C:]

    ext_nbr, mask3 = _edge_tables()
    cpad128 = jnp.pad(coords_pad, ((0, 0), (0, 112)))
    fused = jnp.concatenate([k_pad, v_pad, cpad128], axis=1)
    eidx_ext = ext_nbr.T.reshape(HROWS)
    kb = _sc_gather_half(fused, eidx_ext)

    knn = _knn_idx(coords_pad, coordsT, sq_col, sq_row)
    eidx_knn = knn.T.reshape(HROWS)
    ka = _sc_gather_half(fused, eidx_knn)
    ka3 = ka.reshape(KNN_K, NPAD, FW)
    kb3 = kb.reshape(KNN_K, NPAD, FW)

    wpos_pad = jnp.pad(Wpos.T, ((0, 13), (0, 0)))
    bd = (jnp.arange(C)[:, None] // DIM ==
          jnp.arange(NHEAD)[None, :]).astype(jnp.float32)
    rept = bd.T
    normed = _attention(q_pad, feat_pad, coords_pad, ka3, kb3, mask3,
                        wpos_pad, bpos.reshape(1, NHEAD), bd, rept, Wo.T,
                        gamma.reshape(1, C), beta.reshape(1, C))
    return normed[:n]
